# initial kernel scaffold (unmeasured)
import jax
import jax.numpy as jnp
from jax import lax
from jax.experimental import pallas as pl
from jax.experimental.pallas import tpu as pltpu

N_DEV = 4
FP8 = jnp.float8_e5m2


def kernel(x, w_mat, scale_x, scale_w):
    m_total, k_shard = x.shape
    k_total, n = w_mat.shape
    m_per = m_total // N_DEV

    def body(x_hbm, w_hbm, sx_ref, sw_ref, out_ref,
             xblk, wblk, send_buf, recv_buf, w8,
             x_sems, w_sems, send_sems, recv_sems):
        my = lax.axis_index("i")

        barrier = pltpu.get_barrier_semaphore()
        for off in range(1, N_DEV):
            pl.semaphore_signal(
                barrier, inc=1,
                device_id=((my + off) % N_DEV,),
                device_id_type=pl.DeviceIdType.MESH,
            )
        pl.semaphore_wait(barrier, N_DEV - 1)

        x_copies = []
        for j in range(N_DEV):
            off = j + 1 if j < 3 else 0
            r = (my + off) % N_DEV
            cp = pltpu.make_async_copy(
                x_hbm.at[pl.ds(r * m_per, m_per), :],
                xblk.at[j],
                x_sems.at[j],
            )
            cp.start()
            x_copies.append(cp)

        def w_copy(j, slot):
            k = (my + N_DEV - j) % N_DEV
            return pltpu.make_async_copy(
                w_hbm.at[pl.ds(k * k_shard, k_shard), :],
                wblk.at[slot],
                w_sems.at[j],
            )

        w_copies = [w_copy(j, j % 2) for j in range(N_DEV)]
        w_copies[0].start()
        w_copies[1].start()

        sends = []
        for j in range(3):
            dst = (my + j + 1) % N_DEV
            x_copies[j].wait()
            send_buf[j] = xblk[j].astype(FP8)
            rdma = pltpu.make_async_remote_copy(
                src_ref=send_buf.at[j],
                dst_ref=recv_buf.at[j],
                send_sem=send_sems.at[j],
                recv_sem=recv_sems.at[j],
                device_id=(dst,),
                device_id_type=pl.DeviceIdType.MESH,
            )
            rdma.start()
            sends.append(rdma)

        x_copies[3].wait()
        send_buf[3] = xblk[3].astype(FP8)

        for j in range(N_DEV):
            w_copies[j].wait()
            w8[j] = wblk[j % 2].astype(FP8)
            if j + 2 < N_DEV:
                w_copies[j + 2].start()

        dims = (((1,), (0,)), ((), ()))

        out_ref[...] = lax.dot_general(
            send_buf[3], w8[0], dims, preferred_element_type=jnp.float32)

        for off in (1, 3, 2):
            j = off - 1
            recv = pltpu.make_async_remote_copy(
                src_ref=send_buf.at[j],
                dst_ref=recv_buf.at[j],
                send_sem=send_sems.at[j],
                recv_sem=recv_sems.at[j],
                device_id=(my,),
                device_id_type=pl.DeviceIdType.MESH,
            )
            recv.wait_recv()
            out_ref[...] += lax.dot_general(
                recv_buf[j], w8[off], dims,
                preferred_element_type=jnp.float32)

        s = sx_ref[0] * sw_ref[0]
        y = out_ref[...] * s
        out_ref[...] = y * jax.nn.sigmoid(y)

        for rdma in sends:
            rdma.wait_send()

    return pl.pallas_call(
        body,
        out_shape=jax.ShapeDtypeStruct((m_per, n), jnp.float32),
        in_specs=[
            pl.BlockSpec(memory_space=pltpu.ANY),
            pl.BlockSpec(memory_space=pltpu.ANY),
            pl.BlockSpec(memory_space=pltpu.SMEM),
            pl.BlockSpec(memory_space=pltpu.SMEM),
        ],
        out_specs=pl.BlockSpec(memory_space=pltpu.VMEM),
        scratch_shapes=[
            pltpu.VMEM((N_DEV, m_per, k_shard), jnp.float32),
            pltpu.VMEM((2, k_shard, n), jnp.float32),
            pltpu.VMEM((N_DEV, m_per, k_shard), FP8),
            pltpu.VMEM((3, m_per, k_shard), FP8),
            pltpu.VMEM((N_DEV, k_shard, n), FP8),
            pltpu.SemaphoreType.DMA((N_DEV,)),
            pltpu.SemaphoreType.DMA((N_DEV,)),
            pltpu.SemaphoreType.DMA((3,)),
            pltpu.SemaphoreType.DMA((3,)),
        ],
        compiler_params=pltpu.CompilerParams(collective_id=0),
    )(x, w_mat, scale_x, scale_w)


# baseline (device time: 47282 ns/iter reference)
import jax
import jax.numpy as jnp
from jax import lax
from jax.experimental import pallas as pl
from jax.experimental.pallas import tpu as pltpu

N_DEV = 4
FP8 = jnp.float8_e5m2


def kernel(x, w_mat, scale_x, scale_w):
    m_total, k_shard = x.shape
    k_total, n = w_mat.shape
    m_per = m_total // N_DEV

    def body(x_hbm, w_hbm, sx_ref, sw_ref, out_ref,
             xblk, wblk, send_buf, recv_buf, w8,
             x_sems, w_sems, send_sems, recv_sems):
        my = lax.axis_index("i")

        barrier = pltpu.get_barrier_semaphore()
        for off in range(1, N_DEV):
            pl.semaphore_signal(
                barrier, inc=1,
                device_id=((my + off) % N_DEV,),
                device_id_type=pl.DeviceIdType.MESH,
            )
        pl.semaphore_wait(barrier, N_DEV - 1)

        x_copies = []
        for j in range(N_DEV):
            off = j + 1 if j < 3 else 0
            r = (my + off) % N_DEV
            cp = pltpu.make_async_copy(
                x_hbm.at[pl.ds(r * m_per, m_per), :],
                xblk.at[j],
                x_sems.at[j],
            )
            cp.start()
            x_copies.append(cp)

        def w_copy(j, slot):
            k = (my + N_DEV - j) % N_DEV
            return pltpu.make_async_copy(
                w_hbm.at[pl.ds(k * k_shard, k_shard), :],
                wblk.at[slot],
                w_sems.at[j],
            )

        w_copies = [w_copy(j, j % 2) for j in range(N_DEV)]
        w_copies[0].start()
        w_copies[1].start()

        sends = []
        for j in range(3):
            dst = (my + j + 1) % N_DEV
            x_copies[j].wait()
            send_buf[j] = xblk[j].astype(FP8)
            rdma = pltpu.make_async_remote_copy(
                src_ref=send_buf.at[j],
                dst_ref=recv_buf.at[j],
                send_sem=send_sems.at[j],
                recv_sem=recv_sems.at[j],
                device_id=(dst,),
                device_id_type=pl.DeviceIdType.MESH,
            )
            rdma.start()
            sends.append(rdma)

        x_copies[3].wait()
        send_buf[3] = xblk[3].astype(FP8)

        for j in range(N_DEV):
            w_copies[j].wait()
            w8[j] = wblk[j % 2].astype(FP8)
            if j + 2 < N_DEV:
                w_copies[j + 2].start()

        dims = (((1,), (0,)), ((), ()))

        out_ref[...] = lax.dot_general(
            send_buf[3], w8[0], dims, preferred_element_type=jnp.float32)

        for off in (1, 3, 2):
            j = off - 1
            recv = pltpu.make_async_remote_copy(
                src_ref=send_buf.at[j],
                dst_ref=recv_buf.at[j],
                send_sem=send_sems.at[j],
                recv_sem=recv_sems.at[j],
                device_id=(my,),
                device_id_type=pl.DeviceIdType.MESH,
            )
            recv.wait_recv()
            out_ref[...] += lax.dot_general(
                recv_buf[j], w8[off], dims,
                preferred_element_type=jnp.float32)

        s = sx_ref[0] * sw_ref[0]
        y = out_ref[...] * s
        out_ref[...] = y * jax.nn.sigmoid(y)

        for rdma in sends:
            rdma.wait_send()

    return pl.pallas_call(
        body,
        out_shape=jax.ShapeDtypeStruct((m_per, n), jnp.float32),
        in_specs=[
            pl.BlockSpec(memory_space=pl.ANY),
            pl.BlockSpec(memory_space=pl.ANY),
            pl.BlockSpec(memory_space=pltpu.SMEM),
            pl.BlockSpec(memory_space=pltpu.SMEM),
        ],
        out_specs=pl.BlockSpec(memory_space=pltpu.VMEM),
        scratch_shapes=[
            pltpu.VMEM((N_DEV, m_per, k_shard), jnp.float32),
            pltpu.VMEM((2, k_shard, n), jnp.float32),
            pltpu.VMEM((N_DEV, m_per, k_shard), FP8),
            pltpu.VMEM((3, m_per, k_shard), FP8),
            pltpu.VMEM((N_DEV, k_shard, n), FP8),
            pltpu.SemaphoreType.DMA((N_DEV,)),
            pltpu.SemaphoreType.DMA((N_DEV,)),
            pltpu.SemaphoreType.DMA((3,)),
            pltpu.SemaphoreType.DMA((3,)),
        ],
        compiler_params=pltpu.CompilerParams(
            collective_id=0,
            vmem_limit_bytes=60 * 1024 * 1024,
        ),
    )(x, w_mat, scale_x, scale_w)


# device time: 47263 ns/iter; 1.0004x vs baseline; 1.0004x over previous
import jax
import jax.numpy as jnp
from jax import lax
from jax.experimental import pallas as pl
from jax.experimental.pallas import tpu as pltpu

N_DEV = 4
FP8 = jnp.float8_e5m2


def kernel(x, w_mat, scale_x, scale_w):
    m_total, k_shard = x.shape
    k_total, n = w_mat.shape
    m_per = m_total // N_DEV

    def body(x_hbm, w_hbm, sx_ref, sw_ref, out_hbm,
             xblk, wblk, send_buf, recv_buf, w8, acc,
             x_sems, w_sems, out_sem, send_sems, recv_sems):
        my = lax.axis_index("i")

        with jax.named_scope("barrier"):
            barrier = pltpu.get_barrier_semaphore()
            for off in range(1, N_DEV):
                pl.semaphore_signal(
                    barrier, inc=1,
                    device_id=((my + off) % N_DEV,),
                    device_id_type=pl.DeviceIdType.MESH,
                )
            pl.semaphore_wait(barrier, N_DEV - 1)

        x_copies = []
        for j in range(N_DEV):
            off = j + 1 if j < 3 else 0
            r = (my + off) % N_DEV
            cp = pltpu.make_async_copy(
                x_hbm.at[pl.ds(r * m_per, m_per), :],
                xblk.at[j],
                x_sems.at[j],
            )
            if j < 3:
                cp.start()
            x_copies.append(cp)

        def w_copy(j, slot):
            k = (my + N_DEV - j) % N_DEV
            return pltpu.make_async_copy(
                w_hbm.at[pl.ds(k * k_shard, k_shard), :],
                wblk.at[slot],
                w_sems.at[j],
            )

        w_copies = [w_copy(j, j % 2) for j in range(N_DEV)]

        sends = []
        for j in range(3):
            with jax.named_scope(f"xcast_send#j={j}"):
                dst = (my + j + 1) % N_DEV
                x_copies[j].wait()
                send_buf[j] = xblk[j].astype(FP8)
                rdma = pltpu.make_async_remote_copy(
                    src_ref=send_buf.at[j],
                    dst_ref=recv_buf.at[j],
                    send_sem=send_sems.at[j],
                    recv_sem=recv_sems.at[j],
                    device_id=(dst,),
                    device_id_type=pl.DeviceIdType.MESH,
                )
                rdma.start()
                sends.append(rdma)

        x_copies[3].start()
        w_copies[0].start()
        w_copies[1].start()

        with jax.named_scope("xcast_own"):
            x_copies[3].wait()
            send_buf[3] = xblk[3].astype(FP8)

        for j in range(N_DEV):
            with jax.named_scope(f"wcast#j={j}"):
                w_copies[j].wait()
                w8[j] = wblk[j % 2].astype(FP8)
                if j + 2 < N_DEV:
                    w_copies[j + 2].start()

        dims = (((1,), (0,)), ((), ()))

        with jax.named_scope("gemm_local"):
            acc[...] = lax.dot_general(
                send_buf[3], w8[0], dims, preferred_element_type=jnp.float32)

        for off in (1, 3, 2):
            j = off - 1
            with jax.named_scope(f"wait_recv#off={off}"):
                recv = pltpu.make_async_remote_copy(
                    src_ref=send_buf.at[j],
                    dst_ref=recv_buf.at[j],
                    send_sem=send_sems.at[j],
                    recv_sem=recv_sems.at[j],
                    device_id=(my,),
                    device_id_type=pl.DeviceIdType.MESH,
                )
                recv.wait_recv()
            with jax.named_scope(f"gemm#off={off}"):
                acc[...] += lax.dot_general(
                    recv_buf[j], w8[off], dims,
                    preferred_element_type=jnp.float32)

        with jax.named_scope("epilogue"):
            s = sx_ref[0] * sw_ref[0]
            y = acc[...] * s
            acc[...] = y * jax.nn.sigmoid(y)
            out_cp = pltpu.make_async_copy(acc, out_hbm, out_sem)
            out_cp.start()
            out_cp.wait()

            for rdma in sends:
                rdma.wait_send()

    return pl.pallas_call(
        body,
        out_shape=jax.ShapeDtypeStruct((m_per, n), jnp.float32),
        in_specs=[
            pl.BlockSpec(memory_space=pl.ANY),
            pl.BlockSpec(memory_space=pl.ANY),
            pl.BlockSpec(memory_space=pltpu.SMEM),
            pl.BlockSpec(memory_space=pltpu.SMEM),
        ],
        out_specs=pl.BlockSpec(memory_space=pl.ANY),
        scratch_shapes=[
            pltpu.VMEM((N_DEV, m_per, k_shard), jnp.float32),
            pltpu.VMEM((2, k_shard, n), jnp.float32),
            pltpu.VMEM((N_DEV, m_per, k_shard), FP8),
            pltpu.VMEM((3, m_per, k_shard), FP8),
            pltpu.VMEM((N_DEV, k_shard, n), FP8),
            pltpu.VMEM((m_per, n), jnp.float32),
            pltpu.SemaphoreType.DMA((N_DEV,)),
            pltpu.SemaphoreType.DMA((N_DEV,)),
            pltpu.SemaphoreType.DMA,
            pltpu.SemaphoreType.DMA((3,)),
            pltpu.SemaphoreType.DMA((3,)),
        ],
        compiler_params=pltpu.CompilerParams(
            collective_id=0,
            vmem_limit_bytes=60 * 1024 * 1024,
        ),
    )(x, w_mat, scale_x, scale_w)


# device time: 42140 ns/iter; 1.1220x vs baseline; 1.1216x over previous
import jax
import jax.numpy as jnp
from jax import lax
from jax.experimental import pallas as pl
from jax.experimental.pallas import tpu as pltpu

N_DEV = 4
NQ = 8
FP8 = jnp.float8_e5m2


def kernel(x, w_mat, scale_x, scale_w):
    m_total, k_shard = x.shape
    k_total, n = w_mat.shape
    m_per = m_total // N_DEV

    def body(x_hbm, w_hbm, sx_ref, sw_ref, out_hbm,
             xblk, wblk, send_buf, recv_buf, w8, acc,
             x_sems, w_sems, out_sems, send_sems, recv_sems,
             diag_send_sems, diag_recv_sems):
        my = lax.axis_index("i")

        x_copies = [None] * N_DEV
        for j in range(N_DEV):
            off = j + 1 if j < 3 else 0
            r = (my + off) % N_DEV
            x_copies[j] = pltpu.make_async_copy(
                x_hbm.at[pl.ds(r * m_per, m_per), :],
                xblk.at[j],
                x_sems.at[j],
            )
        SEND_ORDER = (0, 2, 1)
        for j in SEND_ORDER:
            x_copies[j].start()

        with jax.named_scope("barrier"):
            barrier = pltpu.get_barrier_semaphore()
            for off in range(1, N_DEV):
                pl.semaphore_signal(
                    barrier, inc=1,
                    device_id=((my + off) % N_DEV,),
                    device_id_type=pl.DeviceIdType.MESH,
                )
            pl.semaphore_wait(barrier, N_DEV - 1)

        W_ORDER = (0, 1, 3, 2)
        W_SLOT = {0: 0, 1: 1, 3: 0, 2: 1}
        W_NEXT = {0: 3, 1: 2}

        def w_copy(j):
            k = (my + N_DEV - j) % N_DEV
            return pltpu.make_async_copy(
                w_hbm.at[pl.ds(k * k_shard, k_shard), :],
                wblk.at[W_SLOT[j]],
                w_sems.at[j],
            )

        w_copies = [w_copy(j) for j in range(N_DEV)]

        mq = m_per // NQ
        mh = m_per // 2
        sends = []
        for j in SEND_ORDER:
            with jax.named_scope(f"xcast_send#j={j}"):
                dst = (my + j + 1) % N_DEV
                x_copies[j].wait()
                send_buf[j] = xblk[j].astype(FP8)
                if j != 1:
                    for h in range(2):
                        hs = pl.ds(h * mh, mh)
                        sem = j + h
                        rdma = pltpu.make_async_remote_copy(
                            src_ref=send_buf.at[j, hs, :],
                            dst_ref=recv_buf.at[j, hs, :],
                            send_sem=send_sems.at[sem],
                            recv_sem=recv_sems.at[sem],
                            device_id=(dst,),
                            device_id_type=pl.DeviceIdType.MESH,
                        )
                        rdma.start()
                        sends.append(rdma)
                else:
                    for q in range(NQ):
                        qs = pl.ds(q * mq, mq)
                        rdma = pltpu.make_async_remote_copy(
                            src_ref=send_buf.at[1, qs, :],
                            dst_ref=recv_buf.at[1, qs, :],
                            send_sem=diag_send_sems.at[q],
                            recv_sem=diag_recv_sems.at[q],
                            device_id=(dst,),
                            device_id_type=pl.DeviceIdType.MESH,
                        )
                        rdma.start()
                        sends.append(rdma)

        x_copies[3].start()
        w_copies[0].start()
        w_copies[1].start()

        dims = (((1,), (0,)), ((), ()))

        def wait_recv_half(j, h):
            hs = pl.ds(h * mh, mh)
            pltpu.make_async_remote_copy(
                src_ref=send_buf.at[j, hs, :],
                dst_ref=recv_buf.at[j, hs, :],
                send_sem=send_sems.at[j + h],
                recv_sem=recv_sems.at[j + h],
                device_id=(my,),
                device_id_type=pl.DeviceIdType.MESH,
            ).wait_recv()

        def wcast(j):
            with jax.named_scope(f"wcast#j={j}"):
                w_copies[j].wait()
                w8[j] = wblk[W_SLOT[j]].astype(FP8)
                if j in W_NEXT:
                    w_copies[W_NEXT[j]].start()

        def gemm_half(j, h, woff, init=False):
            with jax.named_scope(f"gemm#j={j}h={h}"):
                hs = pl.ds(h * mh, mh)
                part = lax.dot_general(
                    recv_buf[j, hs, :], w8[woff], dims,
                    preferred_element_type=jnp.float32)
                acc[hs, :] = part if init else acc[hs, :] + part

        wcast(0)
        wcast(1)

        for h in range(2):
            with jax.named_scope(f"wait_recv#off=1h={h}"):
                wait_recv_half(0, h)
            gemm_half(0, h, 1, init=True)

        with jax.named_scope("xcast_own"):
            x_copies[3].wait()
            send_buf[3] = xblk[3].astype(FP8)
        with jax.named_scope("gemm_local"):
            acc[...] += lax.dot_general(
                send_buf[3], w8[0], dims, preferred_element_type=jnp.float32)

        wcast(3)
        for h in range(2):
            with jax.named_scope(f"wait_recv#off=3h={h}"):
                wait_recv_half(2, h)
            gemm_half(2, h, 3)
        wcast(2)

        s = sx_ref[0] * sw_ref[0]
        out_cps = []
        for q in range(NQ):
            with jax.named_scope(f"tail#q={q}"):
                qs = pl.ds(q * mq, mq)
                pltpu.make_async_remote_copy(
                    src_ref=send_buf.at[1, qs, :],
                    dst_ref=recv_buf.at[1, qs, :],
                    send_sem=diag_send_sems.at[q],
                    recv_sem=diag_recv_sems.at[q],
                    device_id=(my,),
                    device_id_type=pl.DeviceIdType.MESH,
                ).wait_recv()
                a = acc[qs, :] + lax.dot_general(
                    recv_buf[1, qs, :], w8[2], dims,
                    preferred_element_type=jnp.float32)
                y = a * s
                acc[qs, :] = y * jax.nn.sigmoid(y)
                cp = pltpu.make_async_copy(
                    acc.at[qs, :], out_hbm.at[qs, :], out_sems.at[q])
                cp.start()
                out_cps.append(cp)

        with jax.named_scope("drain"):
            for cp in out_cps:
                cp.wait()
            for rdma in sends:
                rdma.wait_send()

    return pl.pallas_call(
        body,
        out_shape=jax.ShapeDtypeStruct((m_per, n), jnp.float32),
        in_specs=[
            pl.BlockSpec(memory_space=pl.ANY),
            pl.BlockSpec(memory_space=pl.ANY),
            pl.BlockSpec(memory_space=pltpu.SMEM),
            pl.BlockSpec(memory_space=pltpu.SMEM),
        ],
        out_specs=pl.BlockSpec(memory_space=pl.ANY),
        scratch_shapes=[
            pltpu.VMEM((N_DEV, m_per, k_shard), jnp.float32),
            pltpu.VMEM((2, k_shard, n), jnp.float32),
            pltpu.VMEM((N_DEV, m_per, k_shard), FP8),
            pltpu.VMEM((3, m_per, k_shard), FP8),
            pltpu.VMEM((N_DEV, k_shard, n), FP8),
            pltpu.VMEM((m_per, n), jnp.float32),
            pltpu.SemaphoreType.DMA((N_DEV,)),
            pltpu.SemaphoreType.DMA((N_DEV,)),
            pltpu.SemaphoreType.DMA((NQ,)),
            pltpu.SemaphoreType.DMA((4,)),
            pltpu.SemaphoreType.DMA((4,)),
            pltpu.SemaphoreType.DMA((NQ,)),
            pltpu.SemaphoreType.DMA((NQ,)),
        ],
        compiler_params=pltpu.CompilerParams(
            collective_id=0,
            vmem_limit_bytes=60 * 1024 * 1024,
        ),
    )(x, w_mat, scale_x, scale_w)


# device time: 41314 ns/iter; 1.1445x vs baseline; 1.0200x over previous
import jax
import jax.numpy as jnp
from jax import lax
from jax.experimental import pallas as pl
from jax.experimental.pallas import tpu as pltpu

N_DEV = 4
NQ = 4
FP8 = jnp.float8_e5m2


def kernel(x, w_mat, scale_x, scale_w):
    m_total, k_shard = x.shape
    k_total, n = w_mat.shape
    m_per = m_total // N_DEV

    def body(x_hbm, w_hbm, sx_ref, sw_ref, out_hbm,
             xblk, wblk, send_buf, recv_buf, w8, acc,
             x_sems, w_sems, out_sems, send_sems, recv_sems,
             diag_send_sems, diag_recv_sems):
        my = lax.axis_index("i")

        x_copies = [None] * N_DEV
        for j in range(N_DEV):
            off = j + 1 if j < 3 else 0
            r = (my + off) % N_DEV
            x_copies[j] = pltpu.make_async_copy(
                x_hbm.at[pl.ds(r * m_per, m_per), :],
                xblk.at[j],
                x_sems.at[j],
            )
        SEND_ORDER = (0, 2, 1)
        for j in SEND_ORDER:
            x_copies[j].start()

        with jax.named_scope("barrier"):
            barrier = pltpu.get_barrier_semaphore()
            for off in range(1, N_DEV):
                pl.semaphore_signal(
                    barrier, inc=1,
                    device_id=((my + off) % N_DEV,),
                    device_id_type=pl.DeviceIdType.MESH,
                )
            pl.semaphore_wait(barrier, N_DEV - 1)

        W_ORDER = (0, 1, 3, 2)
        W_SLOT = {0: 0, 1: 1, 3: 0, 2: 1}
        W_NEXT = {0: 3, 1: 2}

        def w_copy(j):
            k = (my + N_DEV - j) % N_DEV
            return pltpu.make_async_copy(
                w_hbm.at[pl.ds(k * k_shard, k_shard), :],
                wblk.at[W_SLOT[j]],
                w_sems.at[j],
            )

        w_copies = [w_copy(j) for j in range(N_DEV)]

        mq = m_per // NQ
        mh = m_per // 2
        sends = []
        for j in SEND_ORDER:
            with jax.named_scope(f"xcast_send#j={j}"):
                dst = (my + j + 1) % N_DEV
                x_copies[j].wait()
                send_buf[j] = xblk[j].astype(FP8)
                if j != 1:
                    for h in range(2):
                        hs = pl.ds(h * mh, mh)
                        sem = j + h
                        rdma = pltpu.make_async_remote_copy(
                            src_ref=send_buf.at[j, hs, :],
                            dst_ref=recv_buf.at[j, hs, :],
                            send_sem=send_sems.at[sem],
                            recv_sem=recv_sems.at[sem],
                            device_id=(dst,),
                            device_id_type=pl.DeviceIdType.MESH,
                        )
                        rdma.start()
                        sends.append(rdma)
                else:
                    for q in range(NQ):
                        qs = pl.ds(q * mq, mq)
                        rdma = pltpu.make_async_remote_copy(
                            src_ref=send_buf.at[1, qs, :],
                            dst_ref=recv_buf.at[1, qs, :],
                            send_sem=diag_send_sems.at[q],
                            recv_sem=diag_recv_sems.at[q],
                            device_id=(dst,),
                            device_id_type=pl.DeviceIdType.MESH,
                        )
                        rdma.start()
                        sends.append(rdma)

        x_copies[3].start()
        w_copies[0].start()
        w_copies[1].start()

        dims = (((1,), (0,)), ((), ()))

        def wait_recv_half(j, h):
            hs = pl.ds(h * mh, mh)
            pltpu.make_async_remote_copy(
                src_ref=send_buf.at[j, hs, :],
                dst_ref=recv_buf.at[j, hs, :],
                send_sem=send_sems.at[j + h],
                recv_sem=recv_sems.at[j + h],
                device_id=(my,),
                device_id_type=pl.DeviceIdType.MESH,
            ).wait_recv()

        def wcast(j):
            with jax.named_scope(f"wcast#j={j}"):
                w_copies[j].wait()
                w8[j] = wblk[W_SLOT[j]].astype(FP8)
                if j in W_NEXT:
                    w_copies[W_NEXT[j]].start()

        def gemm_half(j, h, woff, init=False):
            with jax.named_scope(f"gemm#j={j}h={h}"):
                hs = pl.ds(h * mh, mh)
                part = lax.dot_general(
                    recv_buf[j, hs, :], w8[woff], dims,
                    preferred_element_type=jnp.float32)
                acc[hs, :] = part if init else acc[hs, :] + part

        wcast(0)
        wcast(1)

        for h in range(2):
            with jax.named_scope(f"wait_recv#off=1h={h}"):
                wait_recv_half(0, h)
            gemm_half(0, h, 1, init=True)

        with jax.named_scope("xcast_own"):
            x_copies[3].wait()
            send_buf[3] = xblk[3].astype(FP8)
        with jax.named_scope("gemm_local"):
            acc[...] += lax.dot_general(
                send_buf[3], w8[0], dims, preferred_element_type=jnp.float32)

        wcast(3)
        for h in range(2):
            with jax.named_scope(f"wait_recv#off=3h={h}"):
                wait_recv_half(2, h)
            gemm_half(2, h, 3)
        wcast(2)

        s = sx_ref[0] * sw_ref[0]
        out_cps = []
        for q in range(NQ):
            with jax.named_scope(f"tail#q={q}"):
                qs = pl.ds(q * mq, mq)
                pltpu.make_async_remote_copy(
                    src_ref=send_buf.at[1, qs, :],
                    dst_ref=recv_buf.at[1, qs, :],
                    send_sem=diag_send_sems.at[q],
                    recv_sem=diag_recv_sems.at[q],
                    device_id=(my,),
                    device_id_type=pl.DeviceIdType.MESH,
                ).wait_recv()
                a = acc[qs, :] + lax.dot_general(
                    recv_buf[1, qs, :], w8[2], dims,
                    preferred_element_type=jnp.float32)
                y = a * s
                acc[qs, :] = y * jax.nn.sigmoid(y)
                cp = pltpu.make_async_copy(
                    acc.at[qs, :], out_hbm.at[qs, :], out_sems.at[q])
                cp.start()
                out_cps.append(cp)

        with jax.named_scope("drain"):
            for cp in out_cps:
                cp.wait()
            for rdma in sends:
                rdma.wait_send()

    return pl.pallas_call(
        body,
        out_shape=jax.ShapeDtypeStruct((m_per, n), jnp.float32),
        in_specs=[
            pl.BlockSpec(memory_space=pl.ANY),
            pl.BlockSpec(memory_space=pl.ANY),
            pl.BlockSpec(memory_space=pltpu.SMEM),
            pl.BlockSpec(memory_space=pltpu.SMEM),
        ],
        out_specs=pl.BlockSpec(memory_space=pl.ANY),
        scratch_shapes=[
            pltpu.VMEM((N_DEV, m_per, k_shard), jnp.float32),
            pltpu.VMEM((2, k_shard, n), jnp.float32),
            pltpu.VMEM((N_DEV, m_per, k_shard), FP8),
            pltpu.VMEM((3, m_per, k_shard), FP8),
            pltpu.VMEM((N_DEV, k_shard, n), FP8),
            pltpu.VMEM((m_per, n), jnp.float32),
            pltpu.SemaphoreType.DMA((N_DEV,)),
            pltpu.SemaphoreType.DMA((N_DEV,)),
            pltpu.SemaphoreType.DMA((NQ,)),
            pltpu.SemaphoreType.DMA((4,)),
            pltpu.SemaphoreType.DMA((4,)),
            pltpu.SemaphoreType.DMA((NQ,)),
            pltpu.SemaphoreType.DMA((NQ,)),
        ],
        compiler_params=pltpu.CompilerParams(
            collective_id=0,
            vmem_limit_bytes=60 * 1024 * 1024,
        ),
    )(x, w_mat, scale_x, scale_w)
